# speedup vs baseline: 9.6913x; 9.6913x over previous
import functools

import jax
import jax.numpy as jnp
from jax import lax
from jax.experimental import pallas as pl
from jax.experimental.pallas import tpu as pltpu
from jax.experimental.pallas import tpu_sc as plsc

VOCAB = 100000
HIDDEN = 128
MAX_POS = 512
B = 4096
L = 200
EPS = 1e-12

NC = 2
NS = 16
NW = NC * NS

N_ROWS = B * L
ROWS_PER_W = N_ROWS // NW
CHUNK = 512
N_CHUNKS = ROWS_PER_W // CHUNK


def _sc_gather(table, flat_ids):
    mesh = plsc.VectorSubcoreMesh(core_axis_name="c", subcore_axis_name="s")

    @functools.partial(
        pl.kernel,
        out_type=jax.ShapeDtypeStruct((N_ROWS, HIDDEN), jnp.float32),
        mesh=mesh,
        scratch_types=[
            pltpu.VMEM((CHUNK,), jnp.int32),
            pltpu.VMEM((CHUNK, HIDDEN), jnp.float32),
            pltpu.SemaphoreType.DMA,
        ],
    )
    def gather_kernel(table_hbm, ids_hbm, out_hbm, idx_v, rows_v, sem):
        wid = lax.axis_index("s") * NC + lax.axis_index("c")
        base = wid * ROWS_PER_W

        @pl.loop(0, N_CHUNKS)
        def _(c):
            off = base + c * CHUNK
            pltpu.sync_copy(ids_hbm.at[pl.ds(off, CHUNK)], idx_v)
            pltpu.async_copy(table_hbm.at[idx_v], rows_v, sem).wait()
            pltpu.sync_copy(rows_v, out_hbm.at[pl.ds(off, CHUNK)])

    return gather_kernel(table, flat_ids)


def _ln_body(x_ref, pos_ref, g_ref, b_ref, o_ref):
    x = x_ref[...] + pos_ref[...][None, :, :]
    mean = jnp.mean(x, axis=-1, keepdims=True)
    var = jnp.mean(jnp.square(x - mean), axis=-1, keepdims=True)
    normed = (x - mean) * lax.rsqrt(var + EPS)
    o_ref[...] = normed * g_ref[...] + b_ref[...]


def _tc_ln(gathered, pos, gamma, beta):
    RB = 32
    grid = (B // RB,)
    return pl.pallas_call(
        _ln_body,
        grid=grid,
        in_specs=[
            pl.BlockSpec((RB, L, HIDDEN), lambda i: (i, 0, 0)),
            pl.BlockSpec((L, HIDDEN), lambda i: (0, 0)),
            pl.BlockSpec((HIDDEN,), lambda i: (0,)),
            pl.BlockSpec((HIDDEN,), lambda i: (0,)),
        ],
        out_specs=pl.BlockSpec((RB, L, HIDDEN), lambda i: (i, 0, 0)),
        out_shape=jax.ShapeDtypeStruct((B, L, HIDDEN), jnp.float32),
    )(gathered, pos, gamma, beta)


def kernel(input_ids, word_embeddings, position_embeddings, ln_gamma, ln_beta):
    flat_ids = input_ids.reshape(-1)
    gathered = _sc_gather(word_embeddings, flat_ids)
    gathered = gathered.reshape(B, L, HIDDEN)
    pos = position_embeddings[:L]
    return _tc_ln(gathered, pos, ln_gamma, ln_beta)


# speedup vs baseline: 10.7851x; 1.1129x over previous
import functools

import jax
import jax.numpy as jnp
from jax import lax
from jax.experimental import pallas as pl
from jax.experimental.pallas import tpu as pltpu
from jax.experimental.pallas import tpu_sc as plsc

VOCAB = 100000
HIDDEN = 128
MAX_POS = 512
B = 4096
L = 200
EPS = 1e-12

NC = 2
NS = 16
NW = NC * NS

N_XCHUNK = 4
B_C = B // N_XCHUNK
ROWS_C = B_C * L
ROWS_PER_W = ROWS_C // NW
CHUNK = 400
N_CHUNKS = ROWS_PER_W // CHUNK

RB = 32
TC_STEPS = B_C // RB


def _sc_gather(table, flat_ids):
    mesh = plsc.VectorSubcoreMesh(core_axis_name="c", subcore_axis_name="s")

    @functools.partial(
        pl.kernel,
        out_type=jax.ShapeDtypeStruct((ROWS_C, HIDDEN), jnp.float32),
        mesh=mesh,
        scratch_types=[
            pltpu.VMEM((CHUNK,), jnp.int32),
            pltpu.VMEM((CHUNK, HIDDEN), jnp.float32),
            pltpu.SemaphoreType.DMA,
        ],
    )
    def gather_kernel(table_hbm, ids_hbm, out_hbm, idx_v, rows_v, sem):
        wid = lax.axis_index("s") * NC + lax.axis_index("c")
        base = wid * ROWS_PER_W

        @pl.loop(0, N_CHUNKS)
        def _(c):
            off = base + c * CHUNK
            pltpu.sync_copy(ids_hbm.at[pl.ds(off, CHUNK)], idx_v)
            pltpu.async_copy(table_hbm.at[idx_v], rows_v, sem).wait()
            pltpu.sync_copy(rows_v, out_hbm.at[pl.ds(off, CHUNK)])

    return gather_kernel(table, flat_ids)


def _ln_body(prev_ref, x_ref, pos_ref, g_ref, b_ref, o_ref):
    del prev_ref
    x = x_ref[...] + pos_ref[...][None, :, :]
    mean = jnp.mean(x, axis=-1, keepdims=True)
    var = jnp.mean(jnp.square(x - mean), axis=-1, keepdims=True)
    normed = (x - mean) * lax.rsqrt(var + EPS)
    o_ref[...] = normed * g_ref[...] + b_ref[...]


def _tc_ln_chunk(prev_out, gathered, pos, gamma, beta, chunk):
    base = chunk * TC_STEPS
    return pl.pallas_call(
        _ln_body,
        grid=(TC_STEPS,),
        in_specs=[
            pl.BlockSpec((8, 8, HIDDEN), lambda i: (0, 0, 0)),
            pl.BlockSpec((RB, L, HIDDEN), lambda i: (i, 0, 0)),
            pl.BlockSpec((L, HIDDEN), lambda i: (0, 0)),
            pl.BlockSpec((HIDDEN,), lambda i: (0,)),
            pl.BlockSpec((HIDDEN,), lambda i: (0,)),
        ],
        out_specs=pl.BlockSpec((RB, L, HIDDEN), lambda i: (base + i, 0, 0)),
        out_shape=jax.ShapeDtypeStruct((B, L, HIDDEN), jnp.float32),
        input_output_aliases={0: 0},
    )(prev_out, gathered, pos, gamma, beta)


def _tc_ln_first(gathered, pos, gamma, beta):
    return pl.pallas_call(
        lambda x_ref, pos_ref, g_ref, b_ref, o_ref: _ln_body(
            None, x_ref, pos_ref, g_ref, b_ref, o_ref),
        grid=(TC_STEPS,),
        in_specs=[
            pl.BlockSpec((RB, L, HIDDEN), lambda i: (i, 0, 0)),
            pl.BlockSpec((L, HIDDEN), lambda i: (0, 0)),
            pl.BlockSpec((HIDDEN,), lambda i: (0,)),
            pl.BlockSpec((HIDDEN,), lambda i: (0,)),
        ],
        out_specs=pl.BlockSpec((RB, L, HIDDEN), lambda i: (i, 0, 0)),
        out_shape=jax.ShapeDtypeStruct((B, L, HIDDEN), jnp.float32),
    )(gathered, pos, gamma, beta)


def kernel(input_ids, word_embeddings, position_embeddings, ln_gamma, ln_beta):
    flat_ids = input_ids.reshape(-1)
    pos = position_embeddings[:L]

    gathered = [
        _sc_gather(word_embeddings, flat_ids[c * ROWS_C:(c + 1) * ROWS_C])
        .reshape(B_C, L, HIDDEN)
        for c in range(N_XCHUNK)
    ]
    out = _tc_ln_first(gathered[0], pos, ln_gamma, ln_beta)
    for c in range(1, N_XCHUNK):
        out = _tc_ln_chunk(out, gathered[c], pos, ln_gamma, ln_beta, c)
    return out


# speedup vs baseline: 11.4511x; 1.0618x over previous
import functools

import jax
import jax.numpy as jnp
from jax import lax
from jax.experimental import pallas as pl
from jax.experimental.pallas import tpu as pltpu
from jax.experimental.pallas import tpu_sc as plsc

VOCAB = 100000
HIDDEN = 128
MAX_POS = 512
B = 4096
L = 200
EPS = 1e-12

NC = 2
NS = 16
NW = NC * NS

N_XCHUNK = 4
B_C = B // N_XCHUNK
ROWS_C = B_C * L
ROWS_PER_W = ROWS_C // NW
CHUNK = 400
N_CHUNKS = ROWS_PER_W // CHUNK

RB = 64
TC_STEPS = B_C // RB


def _sc_gather(table, flat_ids):
    mesh = plsc.VectorSubcoreMesh(core_axis_name="c", subcore_axis_name="s")

    @functools.partial(
        pl.kernel,
        out_type=jax.ShapeDtypeStruct((ROWS_C, HIDDEN), jnp.float32),
        mesh=mesh,
        scratch_types=[
            pltpu.VMEM((CHUNK,), jnp.int32),
            pltpu.VMEM((CHUNK,), jnp.int32),
            pltpu.VMEM((CHUNK, HIDDEN), jnp.float32),
            pltpu.VMEM((CHUNK, HIDDEN), jnp.float32),
            pltpu.SemaphoreType.DMA,
            pltpu.SemaphoreType.DMA,
            pltpu.SemaphoreType.DMA,
            pltpu.SemaphoreType.DMA,
        ],
    )
    def gather_kernel(table_hbm, ids_hbm, out_hbm,
                      idx0, idx1, rows0, rows1, g0, g1, w0, w1):
        wid = lax.axis_index("s") * NC + lax.axis_index("c")
        base = wid * ROWS_PER_W
        idx = (idx0, idx1)
        rows = (rows0, rows1)
        gsem = (g0, g1)
        wsem = (w0, w1)

        pltpu.sync_copy(ids_hbm.at[pl.ds(base, CHUNK)], idx0)
        pltpu.async_copy(table_hbm.at[idx0], rows0, g0)

        @pl.loop(0, N_CHUNKS // 2)
        def _(p):
            for b in (0, 1):
                c = 2 * p + b
                nb = 1 - b
                @pl.when(c + 1 < N_CHUNKS)
                def _():
                    off_n = base + (c + 1) * CHUNK
                    pltpu.sync_copy(ids_hbm.at[pl.ds(off_n, CHUNK)], idx[nb])
                    @pl.when(c >= 1)
                    def _():
                        pltpu.make_async_copy(
                            rows[nb], out_hbm.at[pl.ds(base, CHUNK)],
                            wsem[nb]).wait()
                    pltpu.async_copy(table_hbm.at[idx[nb]], rows[nb], gsem[nb])

                pltpu.make_async_copy(
                    table_hbm.at[idx[b]], rows[b], gsem[b]).wait()
                pltpu.async_copy(
                    rows[b], out_hbm.at[pl.ds(base + c * CHUNK, CHUNK)],
                    wsem[b])

        pltpu.make_async_copy(
            rows0, out_hbm.at[pl.ds(base, CHUNK)], w0).wait()
        pltpu.make_async_copy(
            rows1, out_hbm.at[pl.ds(base, CHUNK)], w1).wait()

    return gather_kernel(table, flat_ids)


def _ln_math(x_ref, pos_ref, g_ref, b_ref, o_ref):
    x = x_ref[...] + pos_ref[...][None, :, :]
    mean = jnp.mean(x, axis=-1, keepdims=True)
    var = jnp.mean(jnp.square(x - mean), axis=-1, keepdims=True)
    normed = (x - mean) * lax.rsqrt(var + EPS)
    o_ref[...] = normed * g_ref[...] + b_ref[...]


def _ln_body(prev_ref, x_ref, pos_ref, g_ref, b_ref, o_ref):
    del prev_ref
    _ln_math(x_ref, pos_ref, g_ref, b_ref, o_ref)


def _tc_ln_chunk(prev_out, gathered, pos, gamma, beta, chunk):
    base = chunk * TC_STEPS
    return pl.pallas_call(
        _ln_body,
        grid=(TC_STEPS,),
        in_specs=[
            pl.BlockSpec((8, 8, HIDDEN), lambda i: (0, 0, 0)),
            pl.BlockSpec((RB, L, HIDDEN), lambda i: (i, 0, 0)),
            pl.BlockSpec((L, HIDDEN), lambda i: (0, 0)),
            pl.BlockSpec((HIDDEN,), lambda i: (0,)),
            pl.BlockSpec((HIDDEN,), lambda i: (0,)),
        ],
        out_specs=pl.BlockSpec((RB, L, HIDDEN), lambda i: (base + i, 0, 0)),
        out_shape=jax.ShapeDtypeStruct((B, L, HIDDEN), jnp.float32),
        input_output_aliases={0: 0},
    )(prev_out, gathered, pos, gamma, beta)


def _tc_ln_first(gathered, pos, gamma, beta):
    return pl.pallas_call(
        _ln_math,
        grid=(TC_STEPS,),
        in_specs=[
            pl.BlockSpec((RB, L, HIDDEN), lambda i: (i, 0, 0)),
            pl.BlockSpec((L, HIDDEN), lambda i: (0, 0)),
            pl.BlockSpec((HIDDEN,), lambda i: (0,)),
            pl.BlockSpec((HIDDEN,), lambda i: (0,)),
        ],
        out_specs=pl.BlockSpec((RB, L, HIDDEN), lambda i: (i, 0, 0)),
        out_shape=jax.ShapeDtypeStruct((B, L, HIDDEN), jnp.float32),
    )(gathered, pos, gamma, beta)


def kernel(input_ids, word_embeddings, position_embeddings, ln_gamma, ln_beta):
    flat_ids = input_ids.reshape(-1)
    pos = position_embeddings[:L]

    gathered = [
        _sc_gather(word_embeddings, flat_ids[c * ROWS_C:(c + 1) * ROWS_C])
        .reshape(B_C, L, HIDDEN)
        for c in range(N_XCHUNK)
    ]
    out = _tc_ln_first(gathered[0], pos, ln_gamma, ln_beta)
    for c in range(1, N_XCHUNK):
        out = _tc_ln_chunk(out, gathered[c], pos, ln_gamma, ln_beta, c)
    return out
